# baseline (speedup 1.0000x reference)
import jax
import jax.numpy as jnp
from jax import lax
from jax.experimental import pallas as pl
from jax.experimental.pallas import tpu as pltpu
from jax.experimental.pallas import tpu_sc as plsc

N_U = 5000
D = 512
NB = 5120
E_PAD = 81920
NPADR = NB - N_U

NC = 2
NS = 16
NT = NC * NS
DH = D // NC
EPT = E_PAD // NT
EPS = E_PAD // NS
KB = 80
ROWS_PT = NB // NS
ZR = 16

_SC_PARAMS = pltpu.CompilerParams(needs_layout_passes=False)


def _deg_body(src_hbm, dst_hbm, degp_hbm, idxbuf, hist2d, hist, sem):
    c = lax.axis_index("c")
    s = lax.axis_index("s")
    wid = s * NC + c
    lane = lax.iota(jnp.int32, 16)
    ones = jnp.full((16,), 1.0, dtype=jnp.float32)
    zeros16 = jnp.zeros((16,), dtype=jnp.float32)
    lane_off = lane * NB

    for which, arr in ((0, src_hbm), (1, dst_hbm)):
        def zero_body(j, _):
            for r in range(8):
                hist2d[pl.ds(j * 128 + r * 16, 16)] = zeros16
            return _
        lax.fori_loop(0, 16 * NB // 128, zero_body, None)

        pltpu.sync_copy(arr.at[pl.ds(wid * EPT, EPT)], idxbuf)

        def scat_body(j, _):
            idx = idxbuf[pl.ds(j * 16, 16)]
            plsc.addupdate_scatter(hist2d, [lane_off + idx], ones)
            return _
        lax.fori_loop(0, EPT // 16, scat_body, None)

        def red_body(j, _):
            acc = hist2d[pl.ds(j * 16, 16)]
            for r in range(1, 16):
                acc = acc + hist2d[pl.ds(r * NB + j * 16, 16)]
            hist[pl.ds(j * 16, 16)] = acc
            return _
        lax.fori_loop(0, NB // 16, red_body, None)

        pltpu.sync_copy(hist, degp_hbm.at[wid, which])


def _degrees(src_p, dst_p):
    mesh = plsc.VectorSubcoreMesh(core_axis_name="c", subcore_axis_name="s")
    return pl.kernel(
        _deg_body,
        compiler_params=_SC_PARAMS,
        out_type=jax.ShapeDtypeStruct((NT, 2, NB), jnp.float32),
        mesh=mesh,
        scratch_types=[
            pltpu.VMEM((EPT,), jnp.int32),
            pltpu.VMEM((16 * NB,), jnp.float32),
            pltpu.VMEM((NB,), jnp.float32),
            pltpu.SemaphoreType.DMA,
        ],
    )(src_p, dst_p)


def _scale_body(eu_ref, ei_ref, degp_ref, hu_ref, hi_ref, su_ref, si_ref):
    i = pl.program_id(0)
    d = jnp.sum(degp_ref[:, :, pl.ds(i * 512, 512)], axis=0)
    su = lax.rsqrt(jnp.maximum(d[0], 1.0)).reshape(512, 1)
    si = lax.rsqrt(jnp.maximum(d[1], 1.0)).reshape(512, 1)
    su_ref[...] = jnp.broadcast_to(su, (512, 128))
    si_ref[...] = jnp.broadcast_to(si, (512, 128))
    eu = eu_ref[...] * su
    ei = ei_ref[...] * si
    for co in range(NC):
        for k in range(2):
            lo = co * DH + k * 128
            hu_ref[co, :, k, :] = eu[:, lo:lo + 128]
            hi_ref[co, :, k, :] = ei[:, lo:lo + 128]


def _scale(eu_p, ei_p, degp):
    nblk = NB // 512
    emb_spec = pl.BlockSpec((512, D), lambda i: (i, 0))
    tab_spec = pl.BlockSpec((NC, 512, 2, 128), lambda i: (0, i, 0, 0))
    col_spec = pl.BlockSpec((512, 128), lambda i: (i, 0))
    return pl.pallas_call(
        _scale_body,
        grid=(nblk,),
        in_specs=[emb_spec, emb_spec,
                  pl.BlockSpec((NT, 2, NB), lambda i: (0, 0, 0))],
        out_specs=[tab_spec] * 2 + [col_spec] * 2,
        out_shape=[jax.ShapeDtypeStruct((NC, NB, 2, 128), jnp.float32)] * 2
        + [jax.ShapeDtypeStruct((NB, 128), jnp.float32)] * 2,
    )(eu_p, ei_p, degp)


def _segsum_body(src_hbm, dst_hbm, hu, hi, aggb_hbm, aggu_hbm,
                 gidx0, gidx1, sidx0, sidx1, rows0, rows1, zbuf, acc,
                 sem0, sem1):
    c = lax.axis_index("c")
    s = lax.axis_index("s")
    zeros16 = jnp.zeros((16,), dtype=jnp.float32)
    nbatch = EPS // KB

    def zero_body(j, _):
        zbuf[j // 16, (j % 16) // 8, pl.ds(((j % 16) % 8) * 16, 16)] = zeros16
        return _
    lax.fori_loop(0, ZR * 16, zero_body, None)

    for tab, g_hbm, sc_hbm, out_hbm in (
            (hu, src_hbm, dst_hbm, aggb_hbm),
            (hi, dst_hbm, src_hbm, aggu_hbm)):
        for z in range(ROWS_PT // ZR):
            pltpu.sync_copy(zbuf, acc.at[pl.ds(s * ROWS_PT + z * ZR, ZR)])
        plsc.subcore_barrier()

        def issue(b, gi, si, rw, sm):
            base = s * EPS + b * KB
            pltpu.sync_copy(g_hbm.at[pl.ds(base, KB)], gi)
            pltpu.sync_copy(sc_hbm.at[pl.ds(base, KB)], si)
            pltpu.async_copy(tab.at[c].at[gi], rw, sm)

        def drain(gi, si, rw, sm):
            pltpu.make_async_copy(tab.at[c].at[gi], rw, sm).wait()
            pltpu.sync_copy(rw, acc.at[si], add=True)

        issue(0, gidx0, sidx0, rows0, sem0)

        def pair(j, _):
            b = j * 2
            issue(b + 1, gidx1, sidx1, rows1, sem1)
            drain(gidx0, sidx0, rows0, sem0)
            issue(b + 2, gidx0, sidx0, rows0, sem0)
            drain(gidx1, sidx1, rows1, sem1)
            return _
        lax.fori_loop(0, (nbatch - 2) // 2, pair, None)

        issue(nbatch - 1, gidx1, sidx1, rows1, sem1)
        drain(gidx0, sidx0, rows0, sem0)
        drain(gidx1, sidx1, rows1, sem1)
        plsc.subcore_barrier()

        pltpu.sync_copy(acc.at[pl.ds(s * ROWS_PT, ROWS_PT)],
                        out_hbm.at[c, pl.ds(s * ROWS_PT, ROWS_PT)])
        plsc.subcore_barrier()


def _segsum(src_p, dst_p, hu, hi):
    mesh = plsc.VectorSubcoreMesh(core_axis_name="c", subcore_axis_name="s")
    return pl.kernel(
        _segsum_body,
        compiler_params=_SC_PARAMS,
        out_type=(jax.ShapeDtypeStruct((NC, NB, 2, 128), jnp.float32),
                  jax.ShapeDtypeStruct((NC, NB, 2, 128), jnp.float32)),
        mesh=mesh,
        scratch_types=[
            pltpu.VMEM((KB,), jnp.int32),
            pltpu.VMEM((KB,), jnp.int32),
            pltpu.VMEM((KB,), jnp.int32),
            pltpu.VMEM((KB,), jnp.int32),
            pltpu.VMEM((KB, 2, 128), jnp.float32),
            pltpu.VMEM((KB, 2, 128), jnp.float32),
            pltpu.VMEM((ZR, 2, 128), jnp.float32),
            pltpu.VMEM_SHARED((NB, 2, 128), jnp.float32),
            pltpu.SemaphoreType.DMA,
            pltpu.SemaphoreType.DMA,
        ],
    )(src_p, dst_p, hu, hi)


MMB = 1000


def _mm_body(aggb_ref, aggu_ref, su_ref, si_ref,
             wub_ref, bub_ref, wbu_ref, bbu_ref,
             outu_ref, outb_ref, outu2_ref, outb2_ref):
    def mm(agg_ref, w_ref):
        w = w_ref[...].astype(jnp.bfloat16)
        acc = jnp.zeros((MMB, D), dtype=jnp.float32)
        for ci in range(NC):
            for k in range(2):
                lo = ci * DH + k * 128
                acc = acc + jnp.dot(
                    agg_ref[ci, :, k, :].astype(jnp.bfloat16),
                    w[lo:lo + 128, :],
                    preferred_element_type=jnp.float32)
        return acc

    outb = si_ref[:, :1] * mm(aggb_ref, wub_ref) + bub_ref[...]
    outu = su_ref[:, :1] * mm(aggu_ref, wbu_ref) + bbu_ref[...]
    outu_ref[...] = outu
    outb_ref[...] = outb
    outu2_ref[...] = outu
    outb2_ref[...] = outb


def _matmul(aggb, aggu, su, si, W_ub, b_ub, W_bu, b_bu):
    nblk = N_U // MMB
    agg_spec = pl.BlockSpec((NC, MMB, 2, 128), lambda i: (0, i, 0, 0))
    col_spec = pl.BlockSpec((MMB, 128), lambda i: (i, 0))
    w_spec = pl.BlockSpec((D, D), lambda i: (0, 0))
    b_spec = pl.BlockSpec((1, D), lambda i: (0, 0))
    out_spec = pl.BlockSpec((MMB, D), lambda i: (i, 0))
    return pl.pallas_call(
        _mm_body,
        grid=(nblk,),
        in_specs=[agg_spec, agg_spec, col_spec, col_spec,
                  w_spec, b_spec, w_spec, b_spec],
        out_specs=[out_spec] * 4,
        out_shape=[jax.ShapeDtypeStruct((N_U, D), jnp.float32)] * 4,
    )(aggb, aggu, su, si, W_ub, b_ub.reshape(1, D), W_bu, b_bu.reshape(1, D))


def _emit_body(u_ref, b_ref, o1_ref, o2_ref, o3_ref, o4_ref):
    u = u_ref[...]
    b = b_ref[...]
    o1_ref[...] = u
    o2_ref[...] = b
    o3_ref[...] = u
    o4_ref[...] = b


def _emit_outputs(outu_p, outb_p):
    spec = pl.BlockSpec((1000, D), lambda i: (i, 0))
    return pl.pallas_call(
        _emit_body,
        grid=(N_U // 1000,),
        in_specs=[spec, spec],
        out_specs=[spec] * 4,
        out_shape=[jax.ShapeDtypeStruct((N_U, D), jnp.float32)] * 4,
    )(outu_p, outb_p)


def kernel(edge_index, user_ids, item_ids, user_emb, item_emb,
           W_ub, b_ub, W_bu, b_bu):
    src = edge_index[0].astype(jnp.int32)
    dst = edge_index[1].astype(jnp.int32)
    npad = E_PAD - src.shape[0]
    pad = N_U + (jnp.arange(npad, dtype=jnp.int32) % NPADR)
    src_p = jnp.concatenate([src, pad])
    dst_p = jnp.concatenate([dst, pad])
    eu_p = jnp.pad(user_emb, ((0, NB - N_U), (0, 0)))
    ei_p = jnp.pad(item_emb, ((0, NB - N_U), (0, 0)))

    degp = _degrees(src_p, dst_p)
    hu, hi, su, si = _scale(eu_p, ei_p, degp)
    aggb, aggu = _segsum(src_p, dst_p, hu, hi)
    return _matmul(aggb, aggu, su, si, W_ub, b_ub, W_bu, b_bu)


def _slice_body(x_ref, o_ref):
    o_ref[...] = x_ref[...]


def _slice_out(x):
    spec = pl.BlockSpec((1000, D), lambda i: (i, 0))
    return pl.pallas_call(
        _slice_body,
        grid=(N_U // 1000,),
        in_specs=[spec],
        out_specs=spec,
        out_shape=jax.ShapeDtypeStruct((N_U, D), jnp.float32),
    )(x)


# speedup vs baseline: 1.1884x; 1.1884x over previous
import jax
import jax.numpy as jnp
from jax import lax
from jax.experimental import pallas as pl
from jax.experimental.pallas import tpu as pltpu
from jax.experimental.pallas import tpu_sc as plsc

N_U = 5000
D = 512
NB = 5120
E_PAD = 81920
NPADR = NB - N_U

NC = 2
NS = 16
NT = NC * NS
DH = D // NC
EPT = E_PAD // NT
EPS = E_PAD // NS
KB = 64
NBATCH = 5120 // KB
ROWS_PT = NB // NS
ZR = 16

_SC_PARAMS = pltpu.CompilerParams(needs_layout_passes=False)


def _deg_body(src_hbm, dst_hbm, degp_hbm, idxbuf, hist2d, hist, sem):
    c = lax.axis_index("c")
    s = lax.axis_index("s")
    wid = s * NC + c
    lane = lax.iota(jnp.int32, 16)
    ones = jnp.full((16,), 1.0, dtype=jnp.float32)
    zeros16 = jnp.zeros((16,), dtype=jnp.float32)
    lane_off = lane * NB

    for which, arr in ((0, src_hbm), (1, dst_hbm)):
        def zero_body(j, _):
            for r in range(8):
                hist2d[pl.ds(j * 128 + r * 16, 16)] = zeros16
            return _
        lax.fori_loop(0, 16 * NB // 128, zero_body, None)

        pltpu.sync_copy(arr.at[pl.ds(wid * EPT, EPT)], idxbuf)

        def scat_body(j, _):
            idx = idxbuf[pl.ds(j * 16, 16)]
            plsc.addupdate_scatter(hist2d, [lane_off + idx], ones)
            return _
        lax.fori_loop(0, EPT // 16, scat_body, None)

        def red_body(j, _):
            acc = hist2d[pl.ds(j * 16, 16)]
            for r in range(1, 16):
                acc = acc + hist2d[pl.ds(r * NB + j * 16, 16)]
            hist[pl.ds(j * 16, 16)] = acc
            return _
        lax.fori_loop(0, NB // 16, red_body, None)

        pltpu.sync_copy(hist, degp_hbm.at[wid, which])


def _degrees(src_p, dst_p):
    mesh = plsc.VectorSubcoreMesh(core_axis_name="c", subcore_axis_name="s")
    return pl.kernel(
        _deg_body,
        compiler_params=_SC_PARAMS,
        out_type=jax.ShapeDtypeStruct((NT, 2, NB), jnp.float32),
        mesh=mesh,
        scratch_types=[
            pltpu.VMEM((EPT,), jnp.int32),
            pltpu.VMEM((16 * NB,), jnp.float32),
            pltpu.VMEM((NB,), jnp.float32),
            pltpu.SemaphoreType.DMA,
        ],
    )(src_p, dst_p)


def _scale_body(eu_ref, ei_ref, degp_ref, hu_ref, hi_ref, su_ref, si_ref):
    i = pl.program_id(0)
    d = jnp.sum(degp_ref[:, :, pl.ds(i * 512, 512)], axis=0)
    su = lax.rsqrt(jnp.maximum(d[0], 1.0)).reshape(512, 1)
    si = lax.rsqrt(jnp.maximum(d[1], 1.0)).reshape(512, 1)
    su_ref[...] = jnp.broadcast_to(su, (512, 128))
    si_ref[...] = jnp.broadcast_to(si, (512, 128))
    eu = eu_ref[...] * su
    ei = ei_ref[...] * si
    for co in range(NC):
        for k in range(2):
            lo = co * DH + k * 128
            hu_ref[co, :, k, :] = eu[:, lo:lo + 128]
            hi_ref[co, :, k, :] = ei[:, lo:lo + 128]


def _scale(eu_p, ei_p, degp):
    nblk = NB // 512
    emb_spec = pl.BlockSpec((512, D), lambda i: (i, 0))
    tab_spec = pl.BlockSpec((NC, 512, 2, 128), lambda i: (0, i, 0, 0))
    col_spec = pl.BlockSpec((512, 128), lambda i: (i, 0))
    return pl.pallas_call(
        _scale_body,
        grid=(nblk,),
        in_specs=[emb_spec, emb_spec,
                  pl.BlockSpec((NT, 2, NB), lambda i: (0, 0, 0))],
        out_specs=[tab_spec] * 2 + [col_spec] * 2,
        out_shape=[jax.ShapeDtypeStruct((NC, NB, 2, 128), jnp.float32)] * 2
        + [jax.ShapeDtypeStruct((NB, 128), jnp.float32)] * 2,
    )(eu_p, ei_p, degp)


def _segsum_body(src_hbm, dst_hbm, src3_hbm, dst3_hbm, hu, hi,
                 aggb_hbm, aggu_hbm, gall, sall, rows0, rows1, acc,
                 sem0, sem1):
    c = lax.axis_index("c")
    s = lax.axis_index("s")
    zeros16 = jnp.zeros((16,), dtype=jnp.float32)

    for tab, g_hbm, sc_hbm, out_hbm in (
            (hu, src_hbm, dst3_hbm, aggb_hbm),
            (hi, dst_hbm, src3_hbm, aggu_hbm)):
        pltpu.sync_copy(g_hbm.at[pl.ds(s * EPS, EPS)], gall)
        pltpu.sync_copy(sc_hbm.at[s], sall)

        def zero_body(j, _):
            for r in range(8):
                jj = j * 8 + r
                rows0[jj // 16, (jj % 16) // 8,
                      pl.ds(((jj % 16) % 8) * 16, 16)] = zeros16
            return _
        lax.fori_loop(0, KB * 16 // 8, zero_body, None)
        for z in range(ROWS_PT // KB):
            pltpu.sync_copy(rows0, acc.at[pl.ds(s * ROWS_PT + z * KB, KB)])
        plsc.subcore_barrier()

        def issue(b, rw, sm):
            pltpu.async_copy(tab.at[c].at[gall.at[pl.ds(b * KB, KB)]],
                             rw, sm)

        def drain(b, rw, sm):
            pltpu.make_async_copy(tab.at[c].at[gall.at[pl.ds(b * KB, KB)]],
                                  rw, sm).wait()
            pltpu.sync_copy(rw, acc.at[sall.at[b]], add=True)

        issue(0, rows0, sem0)

        def pair(j, _):
            b = j * 2
            issue(b + 1, rows1, sem1)
            drain(b, rows0, sem0)
            issue(b + 2, rows0, sem0)
            drain(b + 1, rows1, sem1)
            return _
        lax.fori_loop(0, (NBATCH - 2) // 2, pair, None)

        issue(NBATCH - 1, rows1, sem1)
        drain(NBATCH - 2, rows0, sem0)
        drain(NBATCH - 1, rows1, sem1)
        plsc.subcore_barrier()

        pltpu.sync_copy(acc.at[pl.ds(s * ROWS_PT, ROWS_PT)],
                        out_hbm.at[c, pl.ds(s * ROWS_PT, ROWS_PT)])
        plsc.subcore_barrier()


def _segsum(src_p, dst_p, src3, dst3, hu, hi):
    mesh = plsc.VectorSubcoreMesh(core_axis_name="c", subcore_axis_name="s")
    return pl.kernel(
        _segsum_body,
        compiler_params=_SC_PARAMS,
        out_type=(jax.ShapeDtypeStruct((NC, NB, 2, 128), jnp.float32),
                  jax.ShapeDtypeStruct((NC, NB, 2, 128), jnp.float32)),
        mesh=mesh,
        scratch_types=[
            pltpu.VMEM((EPS,), jnp.int32),
            pltpu.VMEM((NBATCH, KB), jnp.int32),
            pltpu.VMEM((KB, 2, 128), jnp.float32),
            pltpu.VMEM((KB, 2, 128), jnp.float32),
            pltpu.VMEM_SHARED((NB, 2, 128), jnp.float32),
            pltpu.SemaphoreType.DMA,
            pltpu.SemaphoreType.DMA,
        ],
    )(src_p, dst_p, src3, dst3, hu, hi)


MMB = 1000


def _mm_body(aggb_ref, aggu_ref, su_ref, si_ref,
             wub_ref, bub_ref, wbu_ref, bbu_ref,
             outu_ref, outb_ref, outu2_ref, outb2_ref):
    def mm(agg_ref, w_ref):
        acc = jnp.zeros((MMB, D), dtype=jnp.float32)
        for ci in range(NC):
            for k in range(2):
                lo = ci * DH + k * 128
                acc = acc + jnp.dot(agg_ref[ci, :, k, :],
                                    w_ref[lo:lo + 128, :],
                                    preferred_element_type=jnp.float32)
        return acc

    outb = si_ref[:, :1] * mm(aggb_ref, wub_ref) + bub_ref[...]
    outu = su_ref[:, :1] * mm(aggu_ref, wbu_ref) + bbu_ref[...]
    outu_ref[...] = outu
    outb_ref[...] = outb
    outu2_ref[...] = outu
    outb2_ref[...] = outb


def _matmul(aggb, aggu, su, si, W_ub, b_ub, W_bu, b_bu):
    nblk = N_U // MMB
    agg_spec = pl.BlockSpec((NC, MMB, 2, 128), lambda i: (0, i, 0, 0))
    col_spec = pl.BlockSpec((MMB, 128), lambda i: (i, 0))
    w_spec = pl.BlockSpec((D, D), lambda i: (0, 0))
    b_spec = pl.BlockSpec((1, D), lambda i: (0, 0))
    out_spec = pl.BlockSpec((MMB, D), lambda i: (i, 0))
    return pl.pallas_call(
        _mm_body,
        grid=(nblk,),
        in_specs=[agg_spec, agg_spec, col_spec, col_spec,
                  w_spec, b_spec, w_spec, b_spec],
        out_specs=[out_spec] * 4,
        out_shape=[jax.ShapeDtypeStruct((N_U, D), jnp.float32)] * 4,
    )(aggb, aggu, su, si, W_ub, b_ub.reshape(1, D), W_bu, b_bu.reshape(1, D))


def _emit_body(u_ref, b_ref, o1_ref, o2_ref, o3_ref, o4_ref):
    u = u_ref[...]
    b = b_ref[...]
    o1_ref[...] = u
    o2_ref[...] = b
    o3_ref[...] = u
    o4_ref[...] = b


def _emit_outputs(outu_p, outb_p):
    spec = pl.BlockSpec((1000, D), lambda i: (i, 0))
    return pl.pallas_call(
        _emit_body,
        grid=(N_U // 1000,),
        in_specs=[spec, spec],
        out_specs=[spec] * 4,
        out_shape=[jax.ShapeDtypeStruct((N_U, D), jnp.float32)] * 4,
    )(outu_p, outb_p)


def kernel(edge_index, user_ids, item_ids, user_emb, item_emb,
           W_ub, b_ub, W_bu, b_bu):
    src = edge_index[0].astype(jnp.int32)
    dst = edge_index[1].astype(jnp.int32)
    npad = E_PAD - src.shape[0]
    pad = N_U + (jnp.arange(npad, dtype=jnp.int32) % NPADR)
    src_p = jnp.concatenate([src, pad])
    dst_p = jnp.concatenate([dst, pad])
    eu_p = jnp.pad(user_emb, ((0, NB - N_U), (0, 0)))
    ei_p = jnp.pad(item_emb, ((0, NB - N_U), (0, 0)))

    degp = _degrees(src_p, dst_p)
    hu, hi, su, si = _scale(eu_p, ei_p, degp)
    src3 = src_p.reshape(NS, NBATCH, KB)
    dst3 = dst_p.reshape(NS, NBATCH, KB)
    aggb, aggu = _segsum(src_p, dst_p, src3, dst3, hu, hi)
    return _matmul(aggb, aggu, su, si, W_ub, b_ub, W_bu, b_bu)


def _slice_body(x_ref, o_ref):
    o_ref[...] = x_ref[...]


def _slice_out(x):
    spec = pl.BlockSpec((1000, D), lambda i: (i, 0))
    return pl.pallas_call(
        _slice_body,
        grid=(N_U // 1000,),
        in_specs=[spec],
        out_specs=spec,
        out_shape=jax.ShapeDtypeStruct((N_U, D), jnp.float32),
    )(x)
